# baseline (speedup 1.0000x reference)
import functools

import jax
import jax.numpy as jnp
from jax import lax
from jax.experimental import pallas as pl
from jax.experimental.pallas import tpu as pltpu
from jax.experimental.pallas import tpu_sc as plsc

FG_THRESHOLD = 0.5


def _tc_body(rois_ref, gtt_ref, labels_ref, gidx_ref, *, nb, tn, m):
    lane = jax.lax.broadcasted_iota(jnp.int32, (tn, m), 1)
    for b in range(nb):
        g = gtt_ref[b]
        r = rois_ref[b]
        inter = None
        va = None
        vb = None
        for c in range(3):
            blo = r[:, 1 + c : 2 + c]
            bhi = r[:, 4 + c : 5 + c]
            glo = g[c : c + 1, :]
            ghi = g[3 + c : 4 + c, :]
            d = jnp.maximum(jnp.minimum(bhi, ghi) - jnp.maximum(blo, glo), 0.0)
            inter = d if inter is None else inter * d
            sa = jnp.maximum(bhi - blo, 0.0)
            va = sa if va is None else va * sa
            sb = jnp.maximum(ghi - glo, 0.0)
            vb = sb if vb is None else vb * sb
        union = jnp.maximum(va + vb - inter, 1e-9)
        iou = inter / union
        mx = jnp.max(iou, axis=1, keepdims=True)
        labels_ref[b] = (mx[:, 0] >= FG_THRESHOLD).astype(jnp.int32)
        sel = jnp.where(iou == mx, lane, m)
        amin = jnp.min(sel, axis=1)
        gidx_ref[b] = amin + b * m


def _make_sc_gather(total, d, per):
    mesh = plsc.VectorSubcoreMesh(core_axis_name="c", subcore_axis_name="s")
    info = plsc.get_sparse_core_info()
    nc = info.num_cores

    @functools.partial(
        pl.kernel,
        mesh=mesh,
        out_type=jax.ShapeDtypeStruct((total, d), jnp.float32),
        scratch_types=[
            pltpu.VMEM((per,), jnp.int32),
            pltpu.VMEM((per, d), jnp.float32),
            pltpu.SemaphoreType.DMA,
        ],
        compiler_params=pltpu.CompilerParams(use_tc_tiling_on_sc=False),
    )
    def sc_gather(table_hbm, idx_hbm, out_hbm, idx_v, rows_v, sem):
        wid = lax.axis_index("s") * nc + lax.axis_index("c")
        base = jnp.minimum(wid * per, total - per)
        pltpu.sync_copy(idx_hbm.at[pl.ds(base, per)], idx_v)
        pltpu.async_copy(table_hbm.at[idx_v], rows_v, sem).wait()
        pltpu.sync_copy(rows_v, out_hbm.at[pl.ds(base, per)])

    return sc_gather


def kernel(all_rois, gt_boxes, gt_labels, is_sample):
    nb, n, _ = all_rois.shape
    m = gt_boxes.shape[1]
    tn = 512
    gt_t = jnp.swapaxes(gt_boxes, 1, 2)
    labels, gidx = pl.pallas_call(
        functools.partial(_tc_body, nb=nb, tn=tn, m=m),
        grid=(pl.cdiv(n, tn),),
        in_specs=[
            pl.BlockSpec((nb, tn, 7), lambda i: (0, i, 0)),
            pl.BlockSpec((nb, 6, m), lambda i: (0, 0, 0)),
        ],
        out_specs=[
            pl.BlockSpec((nb, tn), lambda i: (0, i)),
            pl.BlockSpec((nb, tn), lambda i: (0, i)),
        ],
        out_shape=[
            jax.ShapeDtypeStruct((nb, n), jnp.int32),
            jax.ShapeDtypeStruct((nb, n), jnp.int32),
        ],
        compiler_params=pltpu.CompilerParams(
            dimension_semantics=("arbitrary",),
        ),
    )(all_rois, gt_t)

    total = nb * n
    info = plsc.get_sparse_core_info()
    nw = info.num_cores * info.num_subcores
    per = ((total + nw - 1) // nw + 7) // 8 * 8
    table = jnp.pad(gt_boxes.reshape(nb * m, 6), ((0, 0), (0, 10)))
    gathered = _make_sc_gather(total, 16, per)(table, gidx.reshape(total))
    return labels, all_rois, gathered[:, :6].reshape(nb, n, 6)


# speedup vs baseline: 1.8465x; 1.8465x over previous
import functools

import jax
import jax.numpy as jnp
from jax import lax
from jax.experimental import pallas as pl
from jax.experimental.pallas import tpu as pltpu
from jax.experimental.pallas import tpu_sc as plsc

FG_THRESHOLD = 0.5


def _tc_body(roist_ref, gt_ref, labels_ref, gidx_ref, *, nb, tp, m):
    miota = jax.lax.broadcasted_iota(jnp.int32, (m, tp), 0)
    for b in range(nb):
        g = gt_ref[b]
        rt = roist_ref[b]
        inter = None
        va = None
        vb = None
        for c in range(3):
            blo = rt[1 + c : 2 + c, :]
            bhi = rt[4 + c : 5 + c, :]
            glo = g[:, c : c + 1]
            ghi = g[:, 3 + c : 4 + c]
            d = jnp.maximum(jnp.minimum(bhi, ghi) - jnp.maximum(blo, glo), 0.0)
            inter = d if inter is None else inter * d
            sa = jnp.maximum(bhi - blo, 0.0)
            va = sa if va is None else va * sa
            sb = jnp.maximum(ghi - glo, 0.0)
            vb = sb if vb is None else vb * sb
        union = jnp.maximum(va + vb - inter, 1e-9)
        iou = inter / union
        mx = jnp.max(iou, axis=0, keepdims=True)
        labels_ref[b] = (mx[0] >= FG_THRESHOLD).astype(jnp.int32)
        sel = jnp.where(iou == mx, miota, m)
        gidx_ref[b] = jnp.min(sel, axis=0) + b * m


def _make_sc_gather(total, d, per):
    mesh = plsc.VectorSubcoreMesh(core_axis_name="c", subcore_axis_name="s")
    info = plsc.get_sparse_core_info()
    nc = info.num_cores

    @functools.partial(
        pl.kernel,
        mesh=mesh,
        out_type=jax.ShapeDtypeStruct((total, d), jnp.float32),
        scratch_types=[
            pltpu.VMEM((per,), jnp.int32),
            pltpu.VMEM((per, d), jnp.float32),
            pltpu.SemaphoreType.DMA,
        ],
        compiler_params=pltpu.CompilerParams(use_tc_tiling_on_sc=False),
    )
    def sc_gather(table_hbm, idx_hbm, out_hbm, idx_v, rows_v, sem):
        wid = lax.axis_index("s") * nc + lax.axis_index("c")
        base = jnp.minimum(wid * per, total - per)
        pltpu.sync_copy(idx_hbm.at[pl.ds(base, per)], idx_v)
        pltpu.async_copy(table_hbm.at[idx_v], rows_v, sem).wait()
        pltpu.sync_copy(rows_v, out_hbm.at[pl.ds(base, per)])

    return sc_gather


def kernel(all_rois, gt_boxes, gt_labels, is_sample):
    nb, n, _ = all_rois.shape
    m = gt_boxes.shape[1]
    tp = 512
    rois_t = jnp.swapaxes(all_rois, 1, 2)
    labels, gidx = pl.pallas_call(
        functools.partial(_tc_body, nb=nb, tp=tp, m=m),
        grid=(pl.cdiv(n, tp),),
        in_specs=[
            pl.BlockSpec((nb, 7, tp), lambda i: (0, 0, i)),
            pl.BlockSpec((nb, m, 6), lambda i: (0, 0, 0)),
        ],
        out_specs=[
            pl.BlockSpec((nb, tp), lambda i: (0, i)),
            pl.BlockSpec((nb, tp), lambda i: (0, i)),
        ],
        out_shape=[
            jax.ShapeDtypeStruct((nb, n), jnp.int32),
            jax.ShapeDtypeStruct((nb, n), jnp.int32),
        ],
        compiler_params=pltpu.CompilerParams(
            dimension_semantics=("arbitrary",),
        ),
    )(rois_t, gt_boxes)

    total = nb * n
    info = plsc.get_sparse_core_info()
    nw = info.num_cores * info.num_subcores
    per = ((total + nw - 1) // nw + 7) // 8 * 8
    table = jnp.pad(gt_boxes.reshape(nb * m, 6), ((0, 0), (0, 10)))
    gathered = _make_sc_gather(total, 16, per)(table, gidx.reshape(total))
    return labels, all_rois, gathered[:, :6].reshape(nb, n, 6)


# speedup vs baseline: 2.0107x; 1.0890x over previous
import functools

import jax
import jax.numpy as jnp
from jax import lax
from jax.experimental import pallas as pl
from jax.experimental.pallas import tpu as pltpu
from jax.experimental.pallas import tpu_sc as plsc

FG_THRESHOLD = 0.5


def _tc_body(roist_ref, gt_ref, labels_ref, gidx_ref, *, nb, tp, m):
    miota = jax.lax.broadcasted_iota(jnp.int32, (m, tp), 0)
    for b in range(nb):
        g = gt_ref[b]
        rt = roist_ref[b]
        inter = None
        va = None
        vb = None
        for c in range(3):
            blo = rt[1 + c : 2 + c, :]
            bhi = rt[4 + c : 5 + c, :]
            glo = g[:, c : c + 1]
            ghi = g[:, 3 + c : 4 + c]
            d = jnp.maximum(jnp.minimum(bhi, ghi) - jnp.maximum(blo, glo), 0.0)
            inter = d if inter is None else inter * d
            sa = jnp.maximum(bhi - blo, 0.0)
            va = sa if va is None else va * sa
            sb = jnp.maximum(ghi - glo, 0.0)
            vb = sb if vb is None else vb * sb
        union = jnp.maximum(va + vb - inter, 1e-9)
        iou = inter / union
        mx = jnp.max(iou, axis=0, keepdims=True)
        labels_ref[b] = (mx[0] >= FG_THRESHOLD).astype(jnp.int32)
        sel = jnp.where(iou == mx, miota, m)
        gidx_ref[b] = jnp.min(sel, axis=0) + b * m


def _make_sc_gather(total, d, per):
    mesh = plsc.VectorSubcoreMesh(core_axis_name="c", subcore_axis_name="s")
    info = plsc.get_sparse_core_info()
    nc = info.num_cores

    @functools.partial(
        pl.kernel,
        mesh=mesh,
        out_type=jax.ShapeDtypeStruct((total, d), jnp.float32),
        scratch_types=[
            pltpu.VMEM((per,), jnp.int32),
            pltpu.VMEM((per, d), jnp.float32),
            pltpu.SemaphoreType.DMA,
        ],
        compiler_params=pltpu.CompilerParams(use_tc_tiling_on_sc=False),
    )
    def sc_gather(table_hbm, idx_hbm, out_hbm, idx_v, rows_v, sem):
        wid = lax.axis_index("s") * nc + lax.axis_index("c")
        base = jnp.minimum(wid * per, total - per)
        pltpu.sync_copy(idx_hbm.at[pl.ds(base, per)], idx_v)
        pltpu.async_copy(table_hbm.at[idx_v], rows_v, sem).wait()
        pltpu.sync_copy(rows_v, out_hbm.at[pl.ds(base, per)])

    return sc_gather


def kernel(all_rois, gt_boxes, gt_labels, is_sample):
    nb, n, _ = all_rois.shape
    m = gt_boxes.shape[1]
    tp = 1024
    rois_t = jnp.swapaxes(all_rois, 1, 2)
    labels, gidx = pl.pallas_call(
        functools.partial(_tc_body, nb=nb, tp=tp, m=m),
        grid=(pl.cdiv(n, tp),),
        in_specs=[
            pl.BlockSpec((nb, 7, tp), lambda i: (0, 0, i)),
            pl.BlockSpec((nb, m, 6), lambda i: (0, 0, 0)),
        ],
        out_specs=[
            pl.BlockSpec((nb, tp), lambda i: (0, i)),
            pl.BlockSpec((nb, tp), lambda i: (0, i)),
        ],
        out_shape=[
            jax.ShapeDtypeStruct((nb, n), jnp.int32),
            jax.ShapeDtypeStruct((nb, n), jnp.int32),
        ],
        compiler_params=pltpu.CompilerParams(
            dimension_semantics=("arbitrary",),
        ),
    )(rois_t, gt_boxes)

    total = nb * n
    info = plsc.get_sparse_core_info()
    nw = info.num_cores * info.num_subcores
    per = ((total + nw - 1) // nw + 7) // 8 * 8
    table = jnp.pad(gt_boxes.reshape(nb * m, 6), ((0, 0), (0, 10)))
    gathered = _make_sc_gather(total, 16, per)(table, gidx.reshape(total))
    return labels, all_rois, gathered[:, :6].reshape(nb, n, 6)
